# baseline (speedup 1.0000x reference)
import jax
import jax.numpy as jnp
from jax import lax
from jax.experimental import pallas as pl
from jax.experimental.pallas import tpu as pltpu, tpu_sc as plsc

N, C, L = 320000, 128, 10000
NW = 32
CH = 256
NCH = N // CH
RW = 16 + CH
G8 = C // 16
OC = 3 * C
OCF = 3 * C + 1
SPAD = 320
LP = NW * SPAD
STP = 10016


def _run_meta(m, sizes):
    lab2 = m.reshape(NCH, CH)
    first = lab2[:, 0]
    last = lab2[:, -1]
    nr = (last - first + 1).astype(jnp.int32)
    startsL = jnp.concatenate(
        [jnp.zeros((1,), jnp.int32), jnp.cumsum(sizes, dtype=jnp.int32)])
    label_rel = lab2 - first[:, None]
    jj = jnp.arange(CH, dtype=jnp.int32)
    cnt = jnp.sum(label_rel[:, :, None] == jj[None, None, :], axis=1,
                  dtype=jnp.int32)
    ces = jnp.concatenate(
        [first[1:] != last[:-1], jnp.ones((1,), bool)])
    flag = (jj[None, :] < nr[:, None] - 1) \
        | ((jj[None, :] == nr[:, None] - 1) & ces[:, None])
    vals = jnp.where(flag, -cnt, cnt)
    runvals = jnp.concatenate(
        [nr[:, None], jnp.zeros((NCH, 15), jnp.int32), vals], axis=1)

    stp = jnp.concatenate(
        [startsL, jnp.zeros((STP - (L + 1),), jnp.int32)])
    return runvals, stp


def _dynlane(w, lane):
    r = w[0]
    for i in range(1, 16):
        r = lax.select(lane == jnp.int32(i), w[i], r)
    return r


def _init_accs():
    return ([jnp.zeros((16,), jnp.float32) for _ in range(G8)]
            + [jnp.full((16,), jnp.inf, jnp.float32) for _ in range(G8)]
            + [jnp.full((16,), -jnp.inf, jnp.float32) for _ in range(G8)])


def _sc_body(x, m1, m2, rv1, rv2, st1, st2, sz1, sz2, o1, s1, o2, s2,
             rows_v, runs_v, wv, lv, stage, szv, sbuf, rsem, qsem, fsem):
    wid = lax.axis_index("s") * 2 + lax.axis_index("c")

    lb = (jnp.int32(L) * wid) >> 5
    lb2 = (jnp.int32(L) * (wid + 1)) >> 5

    for mi, (rv, out, sz, souts, mlab, stt) in enumerate(
            ((rv1, o1, sz1, s1, m1, st1), (rv2, o2, sz2, s2, m2, st2))):
        off1 = pl.multiple_of(lb & jnp.int32(~15), 16)
        pltpu.sync_copy(stt.at[pl.ds(off1, 16)], wv)
        rs = _dynlane(wv[pl.ds(0, 16)], lb & jnp.int32(15))
        off2 = pl.multiple_of(lb2 & jnp.int32(~15), 16)
        pltpu.sync_copy(stt.at[pl.ds(off2, 16)], wv)
        re = _dynlane(wv[pl.ds(0, 16)], lb2 & jnp.int32(15))
        c0 = rs >> 8
        fo = rs & jnp.int32(CH - 1)
        nch = ((re - 1) >> 8) - c0 + 1
        pltpu.sync_copy(mlab.at[pl.ds(c0 * CH, 16)], lv)
        ro = lb - lv[pl.ds(0, 16)][0]

        def start_chunk(c, b, rv=rv):
            pltpu.make_async_copy(
                x.at[pl.ds(c * CH, CH)],
                rows_v.at[pl.ds(b * CH, CH)], rsem.at[b]).start()
            pltpu.make_async_copy(
                rv.at[pl.ds(c, 1)],
                runs_v.at[pl.ds(b, 1)], qsem.at[b]).start()

        start_chunk(c0, jnp.int32(0))

        def chunk_body(ci, carry, c0=c0, fo=fo, ro=ro, lb=lb, rv=rv, out=out):
            out_j, seg_n = carry[0], carry[1]
            accs = list(carry[2:])
            c = c0 + ci
            cur = ci & jnp.int32(1)
            nxt = (ci + 1) & jnp.int32(1)
            boff = cur * CH
            pltpu.make_async_copy(
                x.at[pl.ds(c * CH, CH)],
                rows_v.at[pl.ds(boff, CH)], rsem.at[cur]).wait()
            pltpu.make_async_copy(
                rv.at[pl.ds(c, 1)],
                runs_v.at[pl.ds(cur, 1)], qsem.at[cur]).wait()

            def _pf(_, c=c, nxt=nxt, rv=rv):
                start_chunk(c + 1, nxt, rv=rv)
                return jnp.int32(0)

            lax.cond(ci + 1 < nch, _pf, lambda _: jnp.int32(0), jnp.int32(0))

            nr = runs_v[cur, pl.ds(0, 16)][0]
            isfirst = ci == 0
            row0 = lax.select(isfirst, fo, jnp.int32(0))
            rlo = lax.select(isfirst, ro, jnp.int32(0))
            ngrp = (nr + jnp.int32(15)) >> 4

            def grp_body(gi, gc, rlo=rlo, lb=lb, out=out, boff=boff, cur=cur):
                row, out_j, seg_n = gc[0], gc[1], gc[2]
                accs = list(gc[3:])
                vals = runs_v[cur, pl.ds(16 + gi * 16, 16)]
                for lane in range(16):
                    v = vals[lane]
                    idx = gi * 16 + lane
                    act = idx >= rlo
                    neg = v < jnp.int32(0)
                    is_last = act & neg
                    cnt = lax.select(
                        act, lax.select(neg, -v, v), jnp.int32(0))

                    def row_pair(i, a, row=row, boff=boff):
                        r = boff + row + 2 * i
                        l0 = [rows_v[r, pl.ds(g * 16, 16)]
                              for g in range(G8)]
                        l1 = [rows_v[r + 1, pl.ds(g * 16, 16)]
                              for g in range(G8)]
                        s = [l0[g] + l1[g] for g in range(G8)]
                        mn = [jnp.minimum(l0[g], l1[g]) for g in range(G8)]
                        mx = [jnp.maximum(l0[g], l1[g]) for g in range(G8)]
                        return ([a[g] + s[g] for g in range(G8)]
                                + [jnp.minimum(a[G8 + g], mn[g])
                                   for g in range(G8)]
                                + [jnp.maximum(a[2 * G8 + g], mx[g])
                                   for g in range(G8)])

                    accs = lax.fori_loop(0, cnt >> 1, row_pair, accs)

                    odd = (cnt & jnp.int32(1)) > 0
                    rt = boff + jnp.maximum(row + cnt - 1, jnp.int32(0))
                    lt = [rows_v[rt, pl.ds(g * 16, 16)] for g in range(G8)]
                    accs = ([jnp.where(odd, accs[g] + lt[g], accs[g])
                             for g in range(G8)]
                            + [jnp.where(odd,
                                         jnp.minimum(accs[G8 + g], lt[g]),
                                         accs[G8 + g]) for g in range(G8)]
                            + [jnp.where(odd,
                                         jnp.maximum(accs[2 * G8 + g],
                                                     lt[g]),
                                         accs[2 * G8 + g])
                               for g in range(G8)])
                    seg_n = seg_n + cnt
                    row = row + cnt

                    def do_flush(a, out_j=out_j, seg_n=seg_n, lb=lb, out=out):
                        slot = out_j & jnp.int32(3)
                        lab = lb + out_j

                        def _w(_):
                            pltpu.make_async_copy(
                                stage.at[pl.ds(slot, 1)],
                                out.at[pl.ds(lab, 1)],
                                fsem.at[slot]).wait()
                            return jnp.int32(0)

                        lax.cond(out_j >= jnp.int32(4), _w,
                                 lambda _: jnp.int32(0), jnp.int32(0))
                        nf = jnp.full((16,), seg_n, jnp.int32) \
                            .astype(jnp.float32)
                        inv = jnp.ones((16,), jnp.float32) / nf
                        for g in range(G8):
                            stage[slot, pl.ds(g * 16, 16)] = a[g] * inv
                        for g in range(G8):
                            stage[slot, pl.ds(C + g * 16, 16)] = a[G8 + g]
                        for g in range(G8):
                            stage[slot, pl.ds(2 * C + g * 16, 16)] = \
                                a[2 * G8 + g]
                        pltpu.make_async_copy(
                            stage.at[pl.ds(slot, 1)],
                            out.at[pl.ds(lab, 1)],
                            fsem.at[slot]).start()
                        return _init_accs()

                    accs = lax.cond(is_last, do_flush, lambda a: list(a),
                                    accs)
                    out_j = lax.select(is_last, out_j + 1, out_j)
                    seg_n = lax.select(is_last, jnp.int32(0), seg_n)
                return (row, out_j, seg_n, *accs)

            gfin = lax.fori_loop(0, ngrp, grp_body,
                                 (row0, out_j, seg_n, *accs))
            return gfin[1:]

        fin = lax.fori_loop(0, nch, chunk_body,
                            (jnp.int32(0), jnp.int32(0), *_init_accs()))
        out_j = fin[0]

        def drain(i, _, lb=lb, out=out):
            sl = i & jnp.int32(3)
            pltpu.make_async_copy(
                stage.at[pl.ds(sl, 1)], out.at[pl.ds(lb, 1)],
                fsem.at[sl]).wait()
            return jnp.int32(0)

        lax.fori_loop(jnp.maximum(out_j - 4, 0), out_j, drain, jnp.int32(0))

        sb = wid * SPAD
        pltpu.sync_copy(sz.at[pl.ds(sb, SPAD)], szv)
        for k in range(SPAD // 16):
            v = szv[pl.ds(k * 16, 16)].astype(jnp.float32)
            sbuf[pl.ds(k * 16, 16)] = jnp.exp(-v) - 0.5
        pltpu.sync_copy(sbuf, souts.at[pl.ds(sb, SPAD)])


def _cat_body(o_ref, s_ref, out_ref):
    out_ref[:, :OC] = o_ref[...]
    out_ref[:, OC:] = s_ref[...]


def kernel(input, cell_1_mask, cell_2_mask, cell_1_sizes, cell_2_sizes):
    rv1, st1 = _run_meta(cell_1_mask, cell_1_sizes)
    rv2, st2 = _run_meta(cell_2_mask, cell_2_sizes)
    pad1 = jnp.ones((LP - L,), jnp.int32)
    sz1 = jnp.concatenate([cell_1_sizes.astype(jnp.int32), pad1])
    sz2 = jnp.concatenate([cell_2_sizes.astype(jnp.int32), pad1])

    mesh = plsc.VectorSubcoreMesh(core_axis_name="c", subcore_axis_name="s")
    o1, s1, o2, s2 = pl.kernel(
        _sc_body,
        out_type=[
            jax.ShapeDtypeStruct((L, OC), jnp.float32),
            jax.ShapeDtypeStruct((LP,), jnp.float32),
            jax.ShapeDtypeStruct((L, OC), jnp.float32),
            jax.ShapeDtypeStruct((LP,), jnp.float32),
        ],
        mesh=mesh,
        scratch_types=[
            pltpu.VMEM((2 * CH, C), jnp.float32),
            pltpu.VMEM((2, RW), jnp.int32),
            pltpu.VMEM((16,), jnp.int32),
            pltpu.VMEM((16,), jnp.int32),
            pltpu.VMEM((4, OC), jnp.float32),
            pltpu.VMEM((SPAD,), jnp.int32),
            pltpu.VMEM((SPAD,), jnp.float32),
            pltpu.SemaphoreType.DMA((2,)),
            pltpu.SemaphoreType.DMA((2,)),
            pltpu.SemaphoreType.DMA((4,)),
        ],
    )(input, cell_1_mask, cell_2_mask, rv1, rv2, st1, st2, sz1, sz2)

    BR = 400
    cat = pl.pallas_call(
        _cat_body,
        grid=(L // BR,),
        in_specs=[pl.BlockSpec((BR, OC), lambda i: (i, 0)),
                  pl.BlockSpec((BR, 1), lambda i: (i, 0))],
        out_specs=pl.BlockSpec((BR, OCF), lambda i: (i, 0)),
        out_shape=jax.ShapeDtypeStruct((L, OCF), jnp.float32),
    )
    r1 = cat(o1, s1[:L, None])
    r2 = cat(o2, s2[:L, None])
    return (r1, r2)


# speedup vs baseline: 1.0741x; 1.0741x over previous
import jax
import jax.numpy as jnp
from jax import lax
from jax.experimental import pallas as pl
from jax.experimental.pallas import tpu as pltpu, tpu_sc as plsc

N, C, L = 320000, 128, 10000
NW = 32
CH = 256
NCH = N // CH
RW = 16 + CH
G8 = C // 16
OC = 3 * C
OCF = 3 * C + 1
SPAD = 320
LP = NW * SPAD
STP = 10016


def _run_meta(m, sizes):
    lab2 = m.reshape(NCH, CH)
    first = lab2[:, 0]
    last = lab2[:, -1]
    nr = (last - first + 1).astype(jnp.int32)
    startsL = jnp.concatenate(
        [jnp.zeros((1,), jnp.int32), jnp.cumsum(sizes, dtype=jnp.int32)])
    label_rel = lab2 - first[:, None]
    jj = jnp.arange(CH, dtype=jnp.int32)
    cnt = jnp.sum(label_rel[:, :, None] == jj[None, None, :], axis=1,
                  dtype=jnp.int32)
    ces = jnp.concatenate(
        [first[1:] != last[:-1], jnp.ones((1,), bool)])
    flag = (jj[None, :] < nr[:, None] - 1) \
        | ((jj[None, :] == nr[:, None] - 1) & ces[:, None])
    vals = jnp.where(flag, -cnt, cnt)
    runvals = jnp.concatenate(
        [nr[:, None], jnp.zeros((NCH, 15), jnp.int32), vals], axis=1)

    stp = jnp.concatenate(
        [startsL, jnp.zeros((STP - (L + 1),), jnp.int32)])
    return runvals, stp


def _dynlane(w, lane):
    r = w[0]
    for i in range(1, 16):
        r = lax.select(lane == jnp.int32(i), w[i], r)
    return r


def _init_accs():
    return ([jnp.zeros((16,), jnp.float32) for _ in range(G8)]
            + [jnp.full((16,), jnp.inf, jnp.float32) for _ in range(G8)]
            + [jnp.full((16,), -jnp.inf, jnp.float32) for _ in range(G8)])


def _sc_body(x, m1, m2, rv1, rv2, st1, st2, sz1, sz2, o1, s1, o2, s2,
             rows_v, runs_v, wv, lv, stage, szv, sbuf, rsem, qsem, fsem):
    wid = lax.axis_index("s") * 2 + lax.axis_index("c")

    lb = (jnp.int32(L) * wid) >> 5
    lb2 = (jnp.int32(L) * (wid + 1)) >> 5

    for mi, (rv, out, sz, souts, mlab, stt) in enumerate(
            ((rv1, o1, sz1, s1, m1, st1), (rv2, o2, sz2, s2, m2, st2))):
        off1 = pl.multiple_of(lb & jnp.int32(~15), 16)
        pltpu.sync_copy(stt.at[pl.ds(off1, 16)], wv)
        rs = _dynlane(wv[pl.ds(0, 16)], lb & jnp.int32(15))
        off2 = pl.multiple_of(lb2 & jnp.int32(~15), 16)
        pltpu.sync_copy(stt.at[pl.ds(off2, 16)], wv)
        re = _dynlane(wv[pl.ds(0, 16)], lb2 & jnp.int32(15))
        c0 = rs >> 8
        fo = rs & jnp.int32(CH - 1)
        nch = ((re - 1) >> 8) - c0 + 1
        pltpu.sync_copy(mlab.at[pl.ds(c0 * CH, 16)], lv)
        ro = lb - lv[pl.ds(0, 16)][0]

        def start_chunk(c, b, rv=rv):
            pltpu.make_async_copy(
                x.at[pl.ds(c * CH, CH)],
                rows_v.at[pl.ds(b * CH, CH)], rsem.at[b]).start()
            pltpu.make_async_copy(
                rv.at[pl.ds(c, 1)],
                runs_v.at[pl.ds(b, 1)], qsem.at[b]).start()

        start_chunk(c0, jnp.int32(0))

        def chunk_body(ci, carry, c0=c0, fo=fo, ro=ro, lb=lb, rv=rv, out=out):
            out_j, seg_n = carry[0], carry[1]
            accs = list(carry[2:])
            c = c0 + ci
            cur = ci & jnp.int32(1)
            nxt = (ci + 1) & jnp.int32(1)
            boff = cur * CH
            pltpu.make_async_copy(
                x.at[pl.ds(c * CH, CH)],
                rows_v.at[pl.ds(boff, CH)], rsem.at[cur]).wait()
            pltpu.make_async_copy(
                rv.at[pl.ds(c, 1)],
                runs_v.at[pl.ds(cur, 1)], qsem.at[cur]).wait()

            def _pf(_, c=c, nxt=nxt, rv=rv):
                start_chunk(c + 1, nxt, rv=rv)
                return jnp.int32(0)

            lax.cond(ci + 1 < nch, _pf, lambda _: jnp.int32(0), jnp.int32(0))

            nr = runs_v[cur, pl.ds(0, 16)][0]
            isfirst = ci == 0
            row0 = lax.select(isfirst, fo, jnp.int32(0))
            rlo = lax.select(isfirst, ro, jnp.int32(0))
            ngrp = (nr + jnp.int32(15)) >> 4

            def grp_body(gi, gc, rlo=rlo, lb=lb, out=out, boff=boff, cur=cur):
                row, out_j, seg_n = gc[0], gc[1], gc[2]
                accs = list(gc[3:])
                vals = runs_v[cur, pl.ds(16 + gi * 16, 16)]
                for lane in range(16):
                    v = vals[lane]
                    idx = gi * 16 + lane
                    act = idx >= rlo
                    neg = v < jnp.int32(0)
                    is_last = act & neg
                    cnt = lax.select(
                        act, lax.select(neg, -v, v), jnp.int32(0))

                    def row_pair(i, a, row=row, boff=boff):
                        r = boff + row + 2 * i
                        na = list(a)
                        for g in range(G8):
                            l0 = rows_v[r, pl.ds(g * 16, 16)]
                            l1 = rows_v[r + 1, pl.ds(g * 16, 16)]
                            na[g] = a[g] + (l0 + l1)
                            na[G8 + g] = jnp.minimum(
                                a[G8 + g], jnp.minimum(l0, l1))
                            na[2 * G8 + g] = jnp.maximum(
                                a[2 * G8 + g], jnp.maximum(l0, l1))
                        return na

                    accs = lax.fori_loop(0, cnt >> 1, row_pair, accs)

                    odd = (cnt & jnp.int32(1)) > 0
                    rt = boff + jnp.maximum(row + cnt - 1, jnp.int32(0))
                    na = list(accs)
                    for g in range(G8):
                        lt = rows_v[rt, pl.ds(g * 16, 16)]
                        na[g] = jnp.where(odd, accs[g] + lt, accs[g])
                        na[G8 + g] = jnp.where(
                            odd, jnp.minimum(accs[G8 + g], lt),
                            accs[G8 + g])
                        na[2 * G8 + g] = jnp.where(
                            odd, jnp.maximum(accs[2 * G8 + g], lt),
                            accs[2 * G8 + g])
                    accs = na
                    seg_n = seg_n + cnt
                    row = row + cnt

                    def do_flush(a, out_j=out_j, seg_n=seg_n, lb=lb, out=out):
                        slot = out_j & jnp.int32(3)
                        lab = lb + out_j

                        def _w(_):
                            pltpu.make_async_copy(
                                stage.at[pl.ds(slot, 1)],
                                out.at[pl.ds(lab, 1)],
                                fsem.at[slot]).wait()
                            return jnp.int32(0)

                        lax.cond(out_j >= jnp.int32(4), _w,
                                 lambda _: jnp.int32(0), jnp.int32(0))
                        nf = jnp.full((16,), seg_n, jnp.int32) \
                            .astype(jnp.float32)
                        inv = jnp.ones((16,), jnp.float32) / nf
                        for g in range(G8):
                            stage[slot, pl.ds(g * 16, 16)] = a[g] * inv
                        for g in range(G8):
                            stage[slot, pl.ds(C + g * 16, 16)] = a[G8 + g]
                        for g in range(G8):
                            stage[slot, pl.ds(2 * C + g * 16, 16)] = \
                                a[2 * G8 + g]
                        pltpu.make_async_copy(
                            stage.at[pl.ds(slot, 1)],
                            out.at[pl.ds(lab, 1)],
                            fsem.at[slot]).start()
                        return _init_accs()

                    accs = lax.cond(is_last, do_flush, lambda a: list(a),
                                    accs)
                    out_j = lax.select(is_last, out_j + 1, out_j)
                    seg_n = lax.select(is_last, jnp.int32(0), seg_n)
                return (row, out_j, seg_n, *accs)

            gfin = lax.fori_loop(0, ngrp, grp_body,
                                 (row0, out_j, seg_n, *accs))
            return gfin[1:]

        fin = lax.fori_loop(0, nch, chunk_body,
                            (jnp.int32(0), jnp.int32(0), *_init_accs()))
        out_j = fin[0]

        def drain(i, _, lb=lb, out=out):
            sl = i & jnp.int32(3)
            pltpu.make_async_copy(
                stage.at[pl.ds(sl, 1)], out.at[pl.ds(lb, 1)],
                fsem.at[sl]).wait()
            return jnp.int32(0)

        lax.fori_loop(jnp.maximum(out_j - 4, 0), out_j, drain, jnp.int32(0))

        sb = wid * SPAD
        pltpu.sync_copy(sz.at[pl.ds(sb, SPAD)], szv)
        for k in range(SPAD // 16):
            v = szv[pl.ds(k * 16, 16)].astype(jnp.float32)
            sbuf[pl.ds(k * 16, 16)] = jnp.exp(-v) - 0.5
        pltpu.sync_copy(sbuf, souts.at[pl.ds(sb, SPAD)])


def _cat_body(o_ref, s_ref, out_ref):
    out_ref[:, :OC] = o_ref[...]
    out_ref[:, OC:] = s_ref[...]


def kernel(input, cell_1_mask, cell_2_mask, cell_1_sizes, cell_2_sizes):
    rv1, st1 = _run_meta(cell_1_mask, cell_1_sizes)
    rv2, st2 = _run_meta(cell_2_mask, cell_2_sizes)
    pad1 = jnp.ones((LP - L,), jnp.int32)
    sz1 = jnp.concatenate([cell_1_sizes.astype(jnp.int32), pad1])
    sz2 = jnp.concatenate([cell_2_sizes.astype(jnp.int32), pad1])

    mesh = plsc.VectorSubcoreMesh(core_axis_name="c", subcore_axis_name="s")
    o1, s1, o2, s2 = pl.kernel(
        _sc_body,
        out_type=[
            jax.ShapeDtypeStruct((L, OC), jnp.float32),
            jax.ShapeDtypeStruct((LP,), jnp.float32),
            jax.ShapeDtypeStruct((L, OC), jnp.float32),
            jax.ShapeDtypeStruct((LP,), jnp.float32),
        ],
        mesh=mesh,
        scratch_types=[
            pltpu.VMEM((2 * CH, C), jnp.float32),
            pltpu.VMEM((2, RW), jnp.int32),
            pltpu.VMEM((16,), jnp.int32),
            pltpu.VMEM((16,), jnp.int32),
            pltpu.VMEM((4, OC), jnp.float32),
            pltpu.VMEM((SPAD,), jnp.int32),
            pltpu.VMEM((SPAD,), jnp.float32),
            pltpu.SemaphoreType.DMA((2,)),
            pltpu.SemaphoreType.DMA((2,)),
            pltpu.SemaphoreType.DMA((4,)),
        ],
    )(input, cell_1_mask, cell_2_mask, rv1, rv2, st1, st2, sz1, sz2)

    BR = 400
    cat = pl.pallas_call(
        _cat_body,
        grid=(L // BR,),
        in_specs=[pl.BlockSpec((BR, OC), lambda i: (i, 0)),
                  pl.BlockSpec((BR, 1), lambda i: (i, 0))],
        out_specs=pl.BlockSpec((BR, OCF), lambda i: (i, 0)),
        out_shape=jax.ShapeDtypeStruct((L, OCF), jnp.float32),
    )
    r1 = cat(o1, s1[:L, None])
    r2 = cat(o2, s2[:L, None])
    return (r1, r2)


# speedup vs baseline: 1.3046x; 1.2146x over previous
import jax
import jax.numpy as jnp
from jax import lax
from jax.experimental import pallas as pl
from jax.experimental.pallas import tpu as pltpu, tpu_sc as plsc

N, C, L = 320000, 128, 10000
NW = 32
CH = 256
NCH = N // CH
RW = 16 + CH
G8 = C // 16
OC = 3 * C
OCF = 3 * C + 1
SPAD = 320
LP = NW * SPAD
STP = 10016


def _run_meta(m, sizes):
    lab2 = m.reshape(NCH, CH)
    first = lab2[:, 0]
    last = lab2[:, -1]
    nr = (last - first + 1).astype(jnp.int32)
    startsL = jnp.concatenate(
        [jnp.zeros((1,), jnp.int32), jnp.cumsum(sizes, dtype=jnp.int32)])
    label_rel = lab2 - first[:, None]
    jj = jnp.arange(CH, dtype=jnp.int32)
    cnt = jnp.sum(label_rel[:, :, None] == jj[None, None, :], axis=1,
                  dtype=jnp.int32)
    ces = jnp.concatenate(
        [first[1:] != last[:-1], jnp.ones((1,), bool)])
    flag = (jj[None, :] < nr[:, None] - 1) \
        | ((jj[None, :] == nr[:, None] - 1) & ces[:, None])
    vals = jnp.where(flag, -cnt, cnt)
    runvals = jnp.concatenate(
        [nr[:, None], jnp.zeros((NCH, 15), jnp.int32), vals], axis=1)

    stp = jnp.concatenate(
        [startsL, jnp.zeros((STP - (L + 1),), jnp.int32)])
    return runvals, stp


def _dynlane(w, lane):
    r = w[0]
    for i in range(1, 16):
        r = lax.select(lane == jnp.int32(i), w[i], r)
    return r


def _init_accs():
    return ([jnp.zeros((16,), jnp.float32) for _ in range(G8)]
            + [jnp.full((16,), jnp.inf, jnp.float32) for _ in range(G8)]
            + [jnp.full((16,), -jnp.inf, jnp.float32) for _ in range(G8)])


def _sc_body(x, m1, m2, rv1, rv2, st1, st2, sz1, sz2, o1, s1, o2, s2,
             rows_v, runs_v, wv, lv, stage, szv, sbuf, rsem, qsem, fsem):
    wid = lax.axis_index("s") * 2 + lax.axis_index("c")

    lb = (jnp.int32(L) * wid) >> 5
    lb2 = (jnp.int32(L) * (wid + 1)) >> 5

    for mi, (rv, out, sz, souts, mlab, stt) in enumerate(
            ((rv1, o1, sz1, s1, m1, st1), (rv2, o2, sz2, s2, m2, st2))):
        off1 = pl.multiple_of(lb & jnp.int32(~15), 16)
        pltpu.sync_copy(stt.at[pl.ds(off1, 16)], wv)
        rs = _dynlane(wv[pl.ds(0, 16)], lb & jnp.int32(15))
        off2 = pl.multiple_of(lb2 & jnp.int32(~15), 16)
        pltpu.sync_copy(stt.at[pl.ds(off2, 16)], wv)
        re = _dynlane(wv[pl.ds(0, 16)], lb2 & jnp.int32(15))
        c0 = rs >> 8
        fo = rs & jnp.int32(CH - 1)
        nch = ((re - 1) >> 8) - c0 + 1
        pltpu.sync_copy(mlab.at[pl.ds(c0 * CH, 16)], lv)
        ro = lb - lv[pl.ds(0, 16)][0]

        def start_chunk(c, b, rv=rv):
            pltpu.make_async_copy(
                x.at[pl.ds(c * CH, CH)],
                rows_v.at[pl.ds(b * CH, CH)], rsem.at[b]).start()
            pltpu.make_async_copy(
                rv.at[pl.ds(c, 1)],
                runs_v.at[pl.ds(b, 1)], qsem.at[b]).start()

        start_chunk(c0, jnp.int32(0))

        def chunk_body(ci, carry, c0=c0, fo=fo, ro=ro, lb=lb, rv=rv, out=out):
            out_j, seg_n = carry[0], carry[1]
            accs = list(carry[2:])
            c = c0 + ci
            cur = ci & jnp.int32(1)
            nxt = (ci + 1) & jnp.int32(1)
            boff = cur * CH
            pltpu.make_async_copy(
                x.at[pl.ds(c * CH, CH)],
                rows_v.at[pl.ds(boff, CH)], rsem.at[cur]).wait()
            pltpu.make_async_copy(
                rv.at[pl.ds(c, 1)],
                runs_v.at[pl.ds(cur, 1)], qsem.at[cur]).wait()

            def _pf(_, c=c, nxt=nxt, rv=rv):
                start_chunk(c + 1, nxt, rv=rv)
                return jnp.int32(0)

            lax.cond(ci + 1 < nch, _pf, lambda _: jnp.int32(0), jnp.int32(0))

            nr = runs_v[cur, pl.ds(0, 16)][0]
            isfirst = ci == 0
            row0 = lax.select(isfirst, fo, jnp.int32(0))
            rlo = lax.select(isfirst, ro, jnp.int32(0))
            ngrp = (nr + jnp.int32(15)) >> 4

            def grp_body(gi, gc, rlo=rlo, lb=lb, out=out, boff=boff, cur=cur):
                row, out_j, seg_n = gc[0], gc[1], gc[2]
                accs = list(gc[3:])
                vals = runs_v[cur, pl.ds(16 + gi * 16, 16)]
                for lane in range(16):
                    v = vals[lane]
                    idx = gi * 16 + lane
                    act = idx >= rlo
                    neg = v < jnp.int32(0)
                    is_last = act & neg
                    cnt = lax.select(
                        act, lax.select(neg, -v, v), jnp.int32(0))

                    def row_body(i, a, row=row, boff=boff):
                        r = boff + row + i
                        ld = [rows_v[r, pl.ds(g * 16, 16)] for g in range(G8)]
                        return ([a[g] + ld[g] for g in range(G8)]
                                + [jnp.minimum(a[G8 + g], ld[g])
                                   for g in range(G8)]
                                + [jnp.maximum(a[2 * G8 + g], ld[g])
                                   for g in range(G8)])

                    accs = lax.fori_loop(0, cnt, row_body, accs)
                    seg_n = seg_n + cnt
                    row = row + cnt

                    def do_flush(a, out_j=out_j, seg_n=seg_n, lb=lb, out=out):
                        slot = out_j & jnp.int32(3)
                        lab = lb + out_j

                        def _w(_):
                            pltpu.make_async_copy(
                                stage.at[pl.ds(slot, 1)],
                                out.at[pl.ds(lab, 1)],
                                fsem.at[slot]).wait()
                            return jnp.int32(0)

                        lax.cond(out_j >= jnp.int32(4), _w,
                                 lambda _: jnp.int32(0), jnp.int32(0))
                        nf = jnp.full((16,), seg_n, jnp.int32) \
                            .astype(jnp.float32)
                        inv = jnp.ones((16,), jnp.float32) / nf
                        for g in range(G8):
                            stage[slot, pl.ds(g * 16, 16)] = a[g] * inv
                        for g in range(G8):
                            stage[slot, pl.ds(C + g * 16, 16)] = a[G8 + g]
                        for g in range(G8):
                            stage[slot, pl.ds(2 * C + g * 16, 16)] = \
                                a[2 * G8 + g]
                        pltpu.make_async_copy(
                            stage.at[pl.ds(slot, 1)],
                            out.at[pl.ds(lab, 1)],
                            fsem.at[slot]).start()
                        return _init_accs()

                    accs = lax.cond(is_last, do_flush, lambda a: list(a),
                                    accs)
                    out_j = lax.select(is_last, out_j + 1, out_j)
                    seg_n = lax.select(is_last, jnp.int32(0), seg_n)
                return (row, out_j, seg_n, *accs)

            gfin = lax.fori_loop(0, ngrp, grp_body,
                                 (row0, out_j, seg_n, *accs))
            return gfin[1:]

        fin = lax.fori_loop(0, nch, chunk_body,
                            (jnp.int32(0), jnp.int32(0), *_init_accs()))
        out_j = fin[0]

        def drain(i, _, lb=lb, out=out):
            sl = i & jnp.int32(3)
            pltpu.make_async_copy(
                stage.at[pl.ds(sl, 1)], out.at[pl.ds(lb, 1)],
                fsem.at[sl]).wait()
            return jnp.int32(0)

        lax.fori_loop(jnp.maximum(out_j - 4, 0), out_j, drain, jnp.int32(0))

        sb = wid * SPAD
        pltpu.sync_copy(sz.at[pl.ds(sb, SPAD)], szv)
        for k in range(SPAD // 16):
            v = szv[pl.ds(k * 16, 16)].astype(jnp.float32)
            sbuf[pl.ds(k * 16, 16)] = jnp.exp(-v) - 0.5
        pltpu.sync_copy(sbuf, souts.at[pl.ds(sb, SPAD)])


def _cat_body(o_ref, s_ref, out_ref):
    out_ref[:, :OC] = o_ref[...]
    out_ref[:, OC:] = s_ref[...]


def kernel(input, cell_1_mask, cell_2_mask, cell_1_sizes, cell_2_sizes):
    rv1, st1 = _run_meta(cell_1_mask, cell_1_sizes)
    rv2, st2 = _run_meta(cell_2_mask, cell_2_sizes)
    pad1 = jnp.ones((LP - L,), jnp.int32)
    sz1 = jnp.concatenate([cell_1_sizes.astype(jnp.int32), pad1])
    sz2 = jnp.concatenate([cell_2_sizes.astype(jnp.int32), pad1])

    mesh = plsc.VectorSubcoreMesh(core_axis_name="c", subcore_axis_name="s")
    o1, s1, o2, s2 = pl.kernel(
        _sc_body,
        out_type=[
            jax.ShapeDtypeStruct((L, OC), jnp.float32),
            jax.ShapeDtypeStruct((LP,), jnp.float32),
            jax.ShapeDtypeStruct((L, OC), jnp.float32),
            jax.ShapeDtypeStruct((LP,), jnp.float32),
        ],
        mesh=mesh,
        scratch_types=[
            pltpu.VMEM((2 * CH, C), jnp.float32),
            pltpu.VMEM((2, RW), jnp.int32),
            pltpu.VMEM((16,), jnp.int32),
            pltpu.VMEM((16,), jnp.int32),
            pltpu.VMEM((4, OC), jnp.float32),
            pltpu.VMEM((SPAD,), jnp.int32),
            pltpu.VMEM((SPAD,), jnp.float32),
            pltpu.SemaphoreType.DMA((2,)),
            pltpu.SemaphoreType.DMA((2,)),
            pltpu.SemaphoreType.DMA((4,)),
        ],
    )(input, cell_1_mask, cell_2_mask, rv1, rv2, st1, st2, sz1, sz2)

    BR = 400
    cat = pl.pallas_call(
        _cat_body,
        grid=(L // BR,),
        in_specs=[pl.BlockSpec((BR, OC), lambda i: (i, 0)),
                  pl.BlockSpec((BR, 1), lambda i: (i, 0))],
        out_specs=pl.BlockSpec((BR, OCF), lambda i: (i, 0)),
        out_shape=jax.ShapeDtypeStruct((L, OCF), jnp.float32),
    )
    r1 = cat(o1, s1[:L, None])
    r2 = cat(o2, s2[:L, None])
    return (r1, r2)


# speedup vs baseline: 1.5807x; 1.2117x over previous
import jax
import jax.numpy as jnp
from jax import lax
from jax.experimental import pallas as pl
from jax.experimental.pallas import tpu as pltpu, tpu_sc as plsc

N, C, L = 320000, 128, 10000
NW = 32
CH = 256
NCH = N // CH
RW = 16 + CH
G8 = C // 16
OC = 3 * C
OCF = 3 * C + 1
SPAD = 320
LP = NW * SPAD
STP = 10016


def _run_meta(m, sizes):
    lab2 = m.reshape(NCH, CH)
    first = lab2[:, 0]
    last = lab2[:, -1]
    nr = (last - first + 1).astype(jnp.int32)
    startsL = jnp.concatenate(
        [jnp.zeros((1,), jnp.int32), jnp.cumsum(sizes, dtype=jnp.int32)])
    label_rel = lab2 - first[:, None]
    jj = jnp.arange(CH, dtype=jnp.int32)
    j16 = jnp.arange(16, dtype=jnp.int32)
    ahi = ((label_rel >> 4)[:, :, None] == j16).astype(jnp.float32)
    alo = ((label_rel & 15)[:, :, None] == j16).astype(jnp.float32)
    cnt = jnp.einsum('cra,crb->cab', ahi, alo,
                     preferred_element_type=jnp.float32) \
        .reshape(NCH, CH).astype(jnp.int32)
    ces = jnp.concatenate(
        [first[1:] != last[:-1], jnp.ones((1,), bool)])
    flag = (jj[None, :] < nr[:, None] - 1) \
        | ((jj[None, :] == nr[:, None] - 1) & ces[:, None])
    vals = jnp.where(flag, -cnt, cnt)
    runvals = jnp.concatenate(
        [nr[:, None], jnp.zeros((NCH, 15), jnp.int32), vals], axis=1)

    stp = jnp.concatenate(
        [startsL, jnp.zeros((STP - (L + 1),), jnp.int32)])
    return runvals, stp


def _dynlane(w, lane):
    r = w[0]
    for i in range(1, 16):
        r = lax.select(lane == jnp.int32(i), w[i], r)
    return r


def _init_accs():
    return ([jnp.zeros((16,), jnp.float32) for _ in range(G8)]
            + [jnp.full((16,), jnp.inf, jnp.float32) for _ in range(G8)]
            + [jnp.full((16,), -jnp.inf, jnp.float32) for _ in range(G8)])


def _sc_body(x, m1, m2, rv1, rv2, st1, st2, sz1, sz2, o1, s1, o2, s2,
             rows_v, runs_v, wv, lv, stage, szv, sbuf, rsem, qsem, fsem):
    wid = lax.axis_index("s") * 2 + lax.axis_index("c")

    lb = (jnp.int32(L) * wid) >> 5
    lb2 = (jnp.int32(L) * (wid + 1)) >> 5

    for mi, (rv, out, sz, souts, mlab, stt) in enumerate(
            ((rv1, o1, sz1, s1, m1, st1), (rv2, o2, sz2, s2, m2, st2))):
        off1 = pl.multiple_of(lb & jnp.int32(~15), 16)
        pltpu.sync_copy(stt.at[pl.ds(off1, 16)], wv)
        rs = _dynlane(wv[pl.ds(0, 16)], lb & jnp.int32(15))
        off2 = pl.multiple_of(lb2 & jnp.int32(~15), 16)
        pltpu.sync_copy(stt.at[pl.ds(off2, 16)], wv)
        re = _dynlane(wv[pl.ds(0, 16)], lb2 & jnp.int32(15))
        c0 = rs >> 8
        fo = rs & jnp.int32(CH - 1)
        nch = ((re - 1) >> 8) - c0 + 1
        pltpu.sync_copy(mlab.at[pl.ds(c0 * CH, 16)], lv)
        ro = lb - lv[pl.ds(0, 16)][0]

        def start_chunk(c, b, rv=rv):
            pltpu.make_async_copy(
                x.at[pl.ds(c * CH, CH)],
                rows_v.at[pl.ds(b * CH, CH)], rsem.at[b]).start()
            pltpu.make_async_copy(
                rv.at[pl.ds(c, 1)],
                runs_v.at[pl.ds(b, 1)], qsem.at[b]).start()

        start_chunk(c0, jnp.int32(0))

        def chunk_body(ci, carry, c0=c0, fo=fo, ro=ro, lb=lb, rv=rv, out=out):
            out_j, seg_n = carry[0], carry[1]
            accs = list(carry[2:])
            c = c0 + ci
            cur = ci & jnp.int32(1)
            nxt = (ci + 1) & jnp.int32(1)
            boff = cur * CH
            pltpu.make_async_copy(
                x.at[pl.ds(c * CH, CH)],
                rows_v.at[pl.ds(boff, CH)], rsem.at[cur]).wait()
            pltpu.make_async_copy(
                rv.at[pl.ds(c, 1)],
                runs_v.at[pl.ds(cur, 1)], qsem.at[cur]).wait()

            def _pf(_, c=c, nxt=nxt, rv=rv):
                start_chunk(c + 1, nxt, rv=rv)
                return jnp.int32(0)

            lax.cond(ci + 1 < nch, _pf, lambda _: jnp.int32(0), jnp.int32(0))

            nr = runs_v[cur, pl.ds(0, 16)][0]
            isfirst = ci == 0
            row0 = lax.select(isfirst, fo, jnp.int32(0))
            rlo = lax.select(isfirst, ro, jnp.int32(0))
            ngrp = (nr + jnp.int32(15)) >> 4

            def grp_body(gi, gc, rlo=rlo, lb=lb, out=out, boff=boff, cur=cur):
                row, out_j, seg_n = gc[0], gc[1], gc[2]
                accs = list(gc[3:])
                vals = runs_v[cur, pl.ds(16 + gi * 16, 16)]
                for lane in range(16):
                    v = vals[lane]
                    idx = gi * 16 + lane
                    act = idx >= rlo
                    neg = v < jnp.int32(0)
                    is_last = act & neg
                    cnt = lax.select(
                        act, lax.select(neg, -v, v), jnp.int32(0))

                    def row_body(i, a, row=row, boff=boff):
                        r = boff + row + i
                        ld = [rows_v[r, pl.ds(g * 16, 16)] for g in range(G8)]
                        return ([a[g] + ld[g] for g in range(G8)]
                                + [jnp.minimum(a[G8 + g], ld[g])
                                   for g in range(G8)]
                                + [jnp.maximum(a[2 * G8 + g], ld[g])
                                   for g in range(G8)])

                    accs = lax.fori_loop(0, cnt, row_body, accs)
                    seg_n = seg_n + cnt
                    row = row + cnt

                    def do_flush(a, out_j=out_j, seg_n=seg_n, lb=lb, out=out):
                        slot = out_j & jnp.int32(3)
                        lab = lb + out_j

                        def _w(_):
                            pltpu.make_async_copy(
                                stage.at[pl.ds(slot, 1)],
                                out.at[pl.ds(lab, 1)],
                                fsem.at[slot]).wait()
                            return jnp.int32(0)

                        lax.cond(out_j >= jnp.int32(4), _w,
                                 lambda _: jnp.int32(0), jnp.int32(0))
                        nf = jnp.full((16,), seg_n, jnp.int32) \
                            .astype(jnp.float32)
                        inv = jnp.ones((16,), jnp.float32) / nf
                        for g in range(G8):
                            stage[slot, pl.ds(g * 16, 16)] = a[g] * inv
                        for g in range(G8):
                            stage[slot, pl.ds(C + g * 16, 16)] = a[G8 + g]
                        for g in range(G8):
                            stage[slot, pl.ds(2 * C + g * 16, 16)] = \
                                a[2 * G8 + g]
                        pltpu.make_async_copy(
                            stage.at[pl.ds(slot, 1)],
                            out.at[pl.ds(lab, 1)],
                            fsem.at[slot]).start()
                        return _init_accs()

                    accs = lax.cond(is_last, do_flush, lambda a: list(a),
                                    accs)
                    out_j = lax.select(is_last, out_j + 1, out_j)
                    seg_n = lax.select(is_last, jnp.int32(0), seg_n)
                return (row, out_j, seg_n, *accs)

            gfin = lax.fori_loop(0, ngrp, grp_body,
                                 (row0, out_j, seg_n, *accs))
            return gfin[1:]

        fin = lax.fori_loop(0, nch, chunk_body,
                            (jnp.int32(0), jnp.int32(0), *_init_accs()))
        out_j = fin[0]

        def drain(i, _, lb=lb, out=out):
            sl = i & jnp.int32(3)
            pltpu.make_async_copy(
                stage.at[pl.ds(sl, 1)], out.at[pl.ds(lb, 1)],
                fsem.at[sl]).wait()
            return jnp.int32(0)

        lax.fori_loop(jnp.maximum(out_j - 4, 0), out_j, drain, jnp.int32(0))

        sb = wid * SPAD
        pltpu.sync_copy(sz.at[pl.ds(sb, SPAD)], szv)
        for k in range(SPAD // 16):
            v = szv[pl.ds(k * 16, 16)].astype(jnp.float32)
            sbuf[pl.ds(k * 16, 16)] = jnp.exp(-v) - 0.5
        pltpu.sync_copy(sbuf, souts.at[pl.ds(sb, SPAD)])


def _cat_body(o_ref, s_ref, out_ref):
    out_ref[:, :OC] = o_ref[...]
    out_ref[:, OC:] = s_ref[...]


def kernel(input, cell_1_mask, cell_2_mask, cell_1_sizes, cell_2_sizes):
    rv1, st1 = _run_meta(cell_1_mask, cell_1_sizes)
    rv2, st2 = _run_meta(cell_2_mask, cell_2_sizes)
    pad1 = jnp.ones((LP - L,), jnp.int32)
    sz1 = jnp.concatenate([cell_1_sizes.astype(jnp.int32), pad1])
    sz2 = jnp.concatenate([cell_2_sizes.astype(jnp.int32), pad1])

    mesh = plsc.VectorSubcoreMesh(core_axis_name="c", subcore_axis_name="s")
    o1, s1, o2, s2 = pl.kernel(
        _sc_body,
        out_type=[
            jax.ShapeDtypeStruct((L, OC), jnp.float32),
            jax.ShapeDtypeStruct((LP,), jnp.float32),
            jax.ShapeDtypeStruct((L, OC), jnp.float32),
            jax.ShapeDtypeStruct((LP,), jnp.float32),
        ],
        mesh=mesh,
        scratch_types=[
            pltpu.VMEM((2 * CH, C), jnp.float32),
            pltpu.VMEM((2, RW), jnp.int32),
            pltpu.VMEM((16,), jnp.int32),
            pltpu.VMEM((16,), jnp.int32),
            pltpu.VMEM((4, OC), jnp.float32),
            pltpu.VMEM((SPAD,), jnp.int32),
            pltpu.VMEM((SPAD,), jnp.float32),
            pltpu.SemaphoreType.DMA((2,)),
            pltpu.SemaphoreType.DMA((2,)),
            pltpu.SemaphoreType.DMA((4,)),
        ],
    )(input, cell_1_mask, cell_2_mask, rv1, rv2, st1, st2, sz1, sz2)

    BR = 400
    cat = pl.pallas_call(
        _cat_body,
        grid=(L // BR,),
        in_specs=[pl.BlockSpec((BR, OC), lambda i: (i, 0)),
                  pl.BlockSpec((BR, 1), lambda i: (i, 0))],
        out_specs=pl.BlockSpec((BR, OCF), lambda i: (i, 0)),
        out_shape=jax.ShapeDtypeStruct((L, OCF), jnp.float32),
    )
    r1 = cat(o1, s1[:L, None])
    r2 = cat(o2, s2[:L, None])
    return (r1, r2)
